# baseline (speedup 1.0000x reference)
import jax
import jax.numpy as jnp
from jax.experimental import pallas as pl
from jax.experimental.pallas import tpu as pltpu


def _encoder_kernel(xcol_ref, w1_ref, b1_ref, w2_ref, b2_ref, w3_ref, b3_ref,
                    ct_ref, cn_ref, out_ref):
    f32 = jnp.float32
    xcol = xcol_ref[0]
    L2 = xcol.shape[0] // 2
    h1 = jnp.dot(xcol, w1_ref[...], preferred_element_type=f32) + b1_ref[...]
    h1 = jnp.maximum(h1, 0.0)
    h1e = h1[:L2, :]
    h1o = h1[L2:, :]
    zrow = jnp.zeros((1, h1.shape[1]), f32)
    sr_o = jnp.concatenate([zrow, h1o[:L2 - 1, :]], axis=0)
    sl_e = jnp.concatenate([h1e[1:, :], zrow], axis=0)
    acts2 = jnp.concatenate([sr_o, h1e, h1o, sl_e], axis=1)
    h2 = jnp.dot(acts2, w2_ref[...], preferred_element_type=f32) + b2_ref[...]
    h2 = jnp.maximum(h2, 0.0)
    sr2 = jnp.concatenate([zrow, h2[:L2 - 1, :]], axis=0)
    sl2 = jnp.concatenate([h2[1:, :], zrow], axis=0)
    acts3 = jnp.concatenate([sr2, h2, sl2], axis=1)
    z = jnp.dot(acts3, w3_ref[...], preferred_element_type=f32) + b3_ref[...]
    d = cn_ref[...] - 2.0 * jnp.dot(z, ct_ref[...], preferred_element_type=f32)
    m = jnp.min(d, axis=1, keepdims=True)
    K = d.shape[1]
    iota = jax.lax.broadcasted_iota(jnp.int32, d.shape, 1)
    idx = jnp.min(jnp.where(d == m, iota, K), axis=1)
    out_ref[0, 0, :] = idx.astype(jnp.int32)


def kernel(x, W1, b1, W2, b2, W3, b3, codebook):
    B, Cin, L = x.shape
    H = W1.shape[0]
    D = W3.shape[0]
    K = codebook.shape[0]
    kw1 = W1.shape[2]
    L1 = L // 2
    L2 = L1 // 2

    xp = jnp.pad(x, ((0, 0), (0, 0), (1, 1)))
    cols = [xp[:, :, t:t + 2 * L1:2] for t in range(kw1)]
    patches = jnp.stack(cols, axis=2)
    patches = patches.reshape(B, Cin * kw1, L1)
    pT = jnp.transpose(patches, (0, 2, 1))
    xcolT = jnp.concatenate([pT[:, 0::2, :], pT[:, 1::2, :]], axis=1)

    W1r = W1.reshape(H, Cin * kw1).T
    W2r = jnp.transpose(W2, (2, 1, 0)).reshape(4 * H, H)
    W3r = jnp.transpose(W3, (2, 1, 0)).reshape(3 * H, D)
    Ct = codebook.T
    cn = jnp.sum(codebook * codebook, axis=1)[None, :]
    b1r, b2r, b3r = b1[None, :], b2[None, :], b3[None, :]

    out = pl.pallas_call(
        _encoder_kernel,
        grid=(B,),
        in_specs=[
            pl.BlockSpec((1, 2 * L2, Cin * kw1), lambda b: (b, 0, 0)),
            pl.BlockSpec((Cin * kw1, H), lambda b: (0, 0)),
            pl.BlockSpec((1, H), lambda b: (0, 0)),
            pl.BlockSpec((4 * H, H), lambda b: (0, 0)),
            pl.BlockSpec((1, H), lambda b: (0, 0)),
            pl.BlockSpec((3 * H, D), lambda b: (0, 0)),
            pl.BlockSpec((1, D), lambda b: (0, 0)),
            pl.BlockSpec((D, K), lambda b: (0, 0)),
            pl.BlockSpec((1, K), lambda b: (0, 0)),
        ],
        out_specs=pl.BlockSpec((1, 1, L2), lambda b: (b, 0, 0)),
        out_shape=jax.ShapeDtypeStruct((B, 1, L2), jnp.int32),
        compiler_params=pltpu.CompilerParams(
            dimension_semantics=("arbitrary",)),
    )(xcolT, W1r, b1r, W2r, b2r, W3r, b3r, Ct, cn)
    return out[:, 0, :]


# speedup vs baseline: 2.4603x; 2.4603x over previous
import jax
import jax.numpy as jnp
from jax.experimental import pallas as pl
from jax.experimental.pallas import tpu as pltpu


def _encoder_kernel(x_ref, w1_ref, b1_ref, w2_ref, b2_ref, w3_ref, b3_ref,
                    ct_ref, cn_ref, out_ref):
    f32 = jnp.float32
    xr = x_ref[0]
    zc = jnp.zeros((xr.shape[0], 1), f32)
    t0 = jnp.concatenate([zc, xr[:, :-1]], axis=1)
    t2 = jnp.concatenate([xr[:, 1:], zc], axis=1)
    t3 = jnp.concatenate([xr[:, 2:], zc, zc], axis=1)
    X16 = jnp.concatenate([t0, xr, t2, t3], axis=0)
    X16T = X16.T
    X4 = X16T.reshape(X16T.shape[0] // 4, 4, X16T.shape[1])
    xcol = jnp.concatenate([X4[:, 0, :], X4[:, 2, :]], axis=0)
    L2 = xcol.shape[0] // 2
    h1 = jnp.dot(xcol, w1_ref[...], preferred_element_type=f32) + b1_ref[...]
    h1 = jnp.maximum(h1, 0.0)
    h1e = h1[:L2, :]
    h1o = h1[L2:, :]
    zrow = jnp.zeros((1, h1.shape[1]), f32)
    sr_o = jnp.concatenate([zrow, h1o[:L2 - 1, :]], axis=0)
    sl_e = jnp.concatenate([h1e[1:, :], zrow], axis=0)
    acts2 = jnp.concatenate([sr_o, h1e, h1o, sl_e], axis=1)
    h2 = jnp.dot(acts2, w2_ref[...], preferred_element_type=f32) + b2_ref[...]
    h2 = jnp.maximum(h2, 0.0)
    sr2 = jnp.concatenate([zrow, h2[:L2 - 1, :]], axis=0)
    sl2 = jnp.concatenate([h2[1:, :], zrow], axis=0)
    acts3 = jnp.concatenate([sr2, h2, sl2], axis=1)
    z = jnp.dot(acts3, w3_ref[...], preferred_element_type=f32) + b3_ref[...]
    d = cn_ref[...] - 2.0 * jnp.dot(z, ct_ref[...], preferred_element_type=f32)
    m = jnp.min(d, axis=1, keepdims=True)
    K = d.shape[1]
    iota = jax.lax.broadcasted_iota(jnp.int32, d.shape, 1)
    idx = jnp.min(jnp.where(d == m, iota, K), axis=1)
    out_ref[0, 0, :] = idx.astype(jnp.int32)


def kernel(x, W1, b1, W2, b2, W3, b3, codebook):
    B, Cin, L = x.shape
    H = W1.shape[0]
    D = W3.shape[0]
    K = codebook.shape[0]
    kw1 = W1.shape[2]
    L1 = L // 2
    L2 = L1 // 2

    W1r = jnp.transpose(W1, (2, 1, 0)).reshape(Cin * kw1, H)
    W2r = jnp.transpose(W2, (2, 1, 0)).reshape(4 * H, H)
    W3r = jnp.transpose(W3, (2, 1, 0)).reshape(3 * H, D)
    Ct = codebook.T
    cn = jnp.sum(codebook * codebook, axis=1)[None, :]
    b1r, b2r, b3r = b1[None, :], b2[None, :], b3[None, :]

    out = pl.pallas_call(
        _encoder_kernel,
        grid=(B,),
        in_specs=[
            pl.BlockSpec((1, Cin, L), lambda b: (b, 0, 0)),
            pl.BlockSpec((Cin * kw1, H), lambda b: (0, 0)),
            pl.BlockSpec((1, H), lambda b: (0, 0)),
            pl.BlockSpec((4 * H, H), lambda b: (0, 0)),
            pl.BlockSpec((1, H), lambda b: (0, 0)),
            pl.BlockSpec((3 * H, D), lambda b: (0, 0)),
            pl.BlockSpec((1, D), lambda b: (0, 0)),
            pl.BlockSpec((D, K), lambda b: (0, 0)),
            pl.BlockSpec((1, K), lambda b: (0, 0)),
        ],
        out_specs=pl.BlockSpec((1, 1, L2), lambda b: (b, 0, 0)),
        out_shape=jax.ShapeDtypeStruct((B, 1, L2), jnp.int32),
        compiler_params=pltpu.CompilerParams(
            dimension_semantics=("arbitrary",)),
    )(x, W1r, b1r, W2r, b2r, W3r, b3r, Ct, cn)
    return out[:, 0, :]


# speedup vs baseline: 2.9344x; 1.1927x over previous
import jax
import jax.numpy as jnp
from jax.experimental import pallas as pl
from jax.experimental.pallas import tpu as pltpu

_NB = 4


def _one_batch(xr, w1, b1, w2, b2, w3, b3, ct, cn):
    f32 = jnp.float32
    zc = jnp.zeros((xr.shape[0], 1), f32)
    t0 = jnp.concatenate([zc, xr[:, :-1]], axis=1)
    t2 = jnp.concatenate([xr[:, 1:], zc], axis=1)
    t3 = jnp.concatenate([xr[:, 2:], zc, zc], axis=1)
    X16 = jnp.concatenate([t0, xr, t2, t3], axis=0)
    X16T = X16.T
    X4 = X16T.reshape(X16T.shape[0] // 4, 4, X16T.shape[1])
    xcol = jnp.concatenate([X4[:, 0, :], X4[:, 2, :]], axis=0)
    L2 = xcol.shape[0] // 2
    h1 = jnp.dot(xcol, w1, preferred_element_type=f32) + b1
    h1 = jnp.maximum(h1, 0.0)
    h1e = h1[:L2, :]
    h1o = h1[L2:, :]
    zrow = jnp.zeros((1, h1.shape[1]), f32)
    sr_o = jnp.concatenate([zrow, h1o[:L2 - 1, :]], axis=0)
    sl_e = jnp.concatenate([h1e[1:, :], zrow], axis=0)
    acts2 = jnp.concatenate([sr_o, h1e, h1o, sl_e], axis=1)
    h2 = jnp.dot(acts2, w2, preferred_element_type=f32) + b2
    h2 = jnp.maximum(h2, 0.0)
    sr2 = jnp.concatenate([zrow, h2[:L2 - 1, :]], axis=0)
    sl2 = jnp.concatenate([h2[1:, :], zrow], axis=0)
    acts3 = jnp.concatenate([sr2, h2, sl2], axis=1)
    z = jnp.dot(acts3, w3, preferred_element_type=f32) + b3
    d = cn - 2.0 * jnp.dot(z, ct, preferred_element_type=f32)
    m = jnp.min(d, axis=1, keepdims=True)
    K = d.shape[1]
    iota = jax.lax.broadcasted_iota(jnp.int32, d.shape, 1)
    return jnp.min(jnp.where(d == m, iota, K), axis=1)


def _encoder_kernel(x_ref, w1_ref, b1_ref, w2_ref, b2_ref, w3_ref, b3_ref,
                    ct_ref, cn_ref, out_ref):
    idxs = [
        _one_batch(x_ref[i], w1_ref[...], b1_ref[...], w2_ref[...],
                   b2_ref[...], w3_ref[...], b3_ref[...], ct_ref[...],
                   cn_ref[...])
        for i in range(_NB)
    ]
    out_ref[...] = jnp.stack(idxs, axis=0).astype(jnp.int32)[None]


def kernel(x, W1, b1, W2, b2, W3, b3, codebook):
    B, Cin, L = x.shape
    H = W1.shape[0]
    D = W3.shape[0]
    K = codebook.shape[0]
    kw1 = W1.shape[2]
    L2 = L // 4

    W1r = jnp.transpose(W1, (2, 1, 0)).reshape(Cin * kw1, H)
    W2r = jnp.transpose(W2, (2, 1, 0)).reshape(4 * H, H)
    W3r = jnp.transpose(W3, (2, 1, 0)).reshape(3 * H, D)
    Ct = codebook.T
    cn = jnp.sum(codebook * codebook, axis=1)[None, :]
    b1r, b2r, b3r = b1[None, :], b2[None, :], b3[None, :]

    out = pl.pallas_call(
        _encoder_kernel,
        grid=(B // _NB,),
        in_specs=[
            pl.BlockSpec((_NB, Cin, L), lambda b: (b, 0, 0)),
            pl.BlockSpec((Cin * kw1, H), lambda b: (0, 0)),
            pl.BlockSpec((1, H), lambda b: (0, 0)),
            pl.BlockSpec((4 * H, H), lambda b: (0, 0)),
            pl.BlockSpec((1, H), lambda b: (0, 0)),
            pl.BlockSpec((3 * H, D), lambda b: (0, 0)),
            pl.BlockSpec((1, D), lambda b: (0, 0)),
            pl.BlockSpec((D, K), lambda b: (0, 0)),
            pl.BlockSpec((1, K), lambda b: (0, 0)),
        ],
        out_specs=pl.BlockSpec((1, _NB, L2), lambda b: (b, 0, 0)),
        out_shape=jax.ShapeDtypeStruct((B // _NB, _NB, L2), jnp.int32),
        compiler_params=pltpu.CompilerParams(
            dimension_semantics=("arbitrary",)),
    )(x, W1r, b1r, W2r, b2r, W3r, b3r, Ct, cn)
    return out.reshape(B, L2)


# speedup vs baseline: 3.0857x; 1.0516x over previous
import jax
import jax.numpy as jnp
from jax.experimental import pallas as pl
from jax.experimental.pallas import tpu as pltpu

_NB = 4


def _tap_stack(xr):
    zc = jnp.zeros((xr.shape[0], 1), jnp.float32)
    t0 = jnp.concatenate([zc, xr[:, :-1]], axis=1)
    t2 = jnp.concatenate([xr[:, 1:], zc], axis=1)
    t3 = jnp.concatenate([xr[:, 2:], zc, zc], axis=1)
    return jnp.concatenate([t0, xr, t2, t3], axis=0)


def _one_batch(xcol, w1, b1, w2, b2, w3, b3, ct, cn):
    f32 = jnp.float32
    L2 = xcol.shape[0] // 2
    h1 = jnp.dot(xcol, w1, preferred_element_type=f32) + b1
    h1 = jnp.maximum(h1, 0.0)
    h1e = h1[:L2, :]
    h1o = h1[L2:, :]
    zrow = jnp.zeros((1, h1.shape[1]), f32)
    sr_o = jnp.concatenate([zrow, h1o[:L2 - 1, :]], axis=0)
    sl_e = jnp.concatenate([h1e[1:, :], zrow], axis=0)
    acts2 = jnp.concatenate([sr_o, h1e, h1o, sl_e], axis=1)
    h2 = jnp.dot(acts2, w2, preferred_element_type=f32) + b2
    h2 = jnp.maximum(h2, 0.0)
    sr2 = jnp.concatenate([zrow, h2[:L2 - 1, :]], axis=0)
    sl2 = jnp.concatenate([h2[1:, :], zrow], axis=0)
    acts3 = jnp.concatenate([sr2, h2, sl2], axis=1)
    z = jnp.dot(acts3, w3, preferred_element_type=f32) + b3
    d = cn - 2.0 * jnp.dot(z, ct, preferred_element_type=f32)
    m = jnp.min(d, axis=1, keepdims=True)
    iota = jax.lax.broadcasted_iota(jnp.int32, d.shape, 1).astype(f32)
    return jnp.min(jnp.where(d == m, iota, jnp.float32(d.shape[1])), axis=1)


def _encoder_kernel(x_ref, w1_ref, b1_ref, w2_ref, b2_ref, w3_ref, b3_ref,
                    ct_ref, cn_ref, out_ref):
    CT = x_ref.shape[1] * 4
    X = jnp.concatenate([_tap_stack(x_ref[i]) for i in range(_NB)], axis=0)
    XT = X.T
    X4 = XT.reshape(XT.shape[0] // 4, 4, XT.shape[1])
    Xe, Xo = X4[:, 0, :], X4[:, 2, :]
    idxs = [
        _one_batch(
            jnp.concatenate([Xe[:, i * CT:(i + 1) * CT],
                             Xo[:, i * CT:(i + 1) * CT]], axis=0),
            w1_ref[...], b1_ref[...], w2_ref[...], b2_ref[...], w3_ref[...],
            b3_ref[...], ct_ref[...], cn_ref[...])
        for i in range(_NB)
    ]
    out_ref[...] = jnp.stack(idxs, axis=0).astype(jnp.int32)[None]


def kernel(x, W1, b1, W2, b2, W3, b3, codebook):
    B, Cin, L = x.shape
    H = W1.shape[0]
    D = W3.shape[0]
    K = codebook.shape[0]
    kw1 = W1.shape[2]
    L2 = L // 4

    W1r = jnp.transpose(W1, (2, 1, 0)).reshape(Cin * kw1, H)
    W2r = jnp.transpose(W2, (2, 1, 0)).reshape(4 * H, H)
    W3r = jnp.transpose(W3, (2, 1, 0)).reshape(3 * H, D)
    Ct = codebook.T
    cn = jnp.sum(codebook * codebook, axis=1)[None, :]
    b1r, b2r, b3r = b1[None, :], b2[None, :], b3[None, :]

    out = pl.pallas_call(
        _encoder_kernel,
        grid=(B // _NB,),
        in_specs=[
            pl.BlockSpec((_NB, Cin, L), lambda b: (b, 0, 0)),
            pl.BlockSpec((Cin * kw1, H), lambda b: (0, 0)),
            pl.BlockSpec((1, H), lambda b: (0, 0)),
            pl.BlockSpec((4 * H, H), lambda b: (0, 0)),
            pl.BlockSpec((1, H), lambda b: (0, 0)),
            pl.BlockSpec((3 * H, D), lambda b: (0, 0)),
            pl.BlockSpec((1, D), lambda b: (0, 0)),
            pl.BlockSpec((D, K), lambda b: (0, 0)),
            pl.BlockSpec((1, K), lambda b: (0, 0)),
        ],
        out_specs=pl.BlockSpec((1, _NB, L2), lambda b: (b, 0, 0)),
        out_shape=jax.ShapeDtypeStruct((B // _NB, _NB, L2), jnp.int32),
        compiler_params=pltpu.CompilerParams(
            dimension_semantics=("arbitrary",)),
    )(x, W1r, b1r, W2r, b2r, W3r, b3r, Ct, cn)
    return out.reshape(B, L2)


# speedup vs baseline: 3.2412x; 1.0504x over previous
import jax
import jax.numpy as jnp
from jax.experimental import pallas as pl
from jax.experimental.pallas import tpu as pltpu

_NB = 4


def _tap_stack(xr):
    zc = jnp.zeros((xr.shape[0], 1), jnp.float32)
    t0 = jnp.concatenate([zc, xr[:, :-1]], axis=1)
    t2 = jnp.concatenate([xr[:, 1:], zc], axis=1)
    t3 = jnp.concatenate([xr[:, 2:], zc, zc], axis=1)
    return jnp.concatenate([t0, xr, t2, t3], axis=0)


def _encoder_kernel(x_ref, w1_ref, b1_ref, w2_ref, b2_ref, w3_ref, b3_ref,
                    ct_ref, cn_ref, out_ref):
    f32 = jnp.float32
    NB = x_ref.shape[0]
    CT = x_ref.shape[1] * 4
    L2 = x_ref.shape[2] // 4
    X = jnp.concatenate([_tap_stack(x_ref[i]) for i in range(NB)], axis=0)
    XT = X.T
    X4 = XT.reshape(XT.shape[0] // 4, 4, XT.shape[1])
    Xe, Xo = X4[:, 0, :], X4[:, 2, :]
    xcol = jnp.concatenate(
        [Xe[:, i * CT:(i + 1) * CT] for i in range(NB)]
        + [Xo[:, i * CT:(i + 1) * CT] for i in range(NB)], axis=0)
    h1 = jnp.dot(xcol, w1_ref[...], preferred_element_type=f32) + b1_ref[...]
    h1 = jnp.maximum(h1, 0.0)
    HE = h1[:NB * L2, :]
    HO = h1[NB * L2:, :]
    zrow = jnp.zeros((1, h1.shape[1]), f32)
    sro_parts, sle_parts = [], []
    for i in range(NB):
        sro_parts += [zrow, HO[i * L2:(i + 1) * L2 - 1, :]]
        sle_parts += [HE[i * L2 + 1:(i + 1) * L2, :], zrow]
    SRO = jnp.concatenate(sro_parts, axis=0)
    SLE = jnp.concatenate(sle_parts, axis=0)
    H = h1.shape[1]
    w2 = w2_ref[...]
    h2 = (jnp.dot(SRO, w2[:H], preferred_element_type=f32)
          + jnp.dot(HE, w2[H:2 * H], preferred_element_type=f32)
          + jnp.dot(HO, w2[2 * H:3 * H], preferred_element_type=f32)
          + jnp.dot(SLE, w2[3 * H:], preferred_element_type=f32)) + b2_ref[...]
    h2 = jnp.maximum(h2, 0.0)
    sr2_parts, sl2_parts = [], []
    for i in range(NB):
        sr2_parts += [zrow, h2[i * L2:(i + 1) * L2 - 1, :]]
        sl2_parts += [h2[i * L2 + 1:(i + 1) * L2, :], zrow]
    SR2 = jnp.concatenate(sr2_parts, axis=0)
    SL2 = jnp.concatenate(sl2_parts, axis=0)
    w3 = w3_ref[...]
    z = (jnp.dot(SR2, w3[:H], preferred_element_type=f32)
         + jnp.dot(h2, w3[H:2 * H], preferred_element_type=f32)
         + jnp.dot(SL2, w3[2 * H:], preferred_element_type=f32)) + b3_ref[...]
    d = cn_ref[...] - 2.0 * jnp.dot(z, ct_ref[...], preferred_element_type=f32)
    idxs = [jnp.argmin(d[i * L2:(i + 1) * L2, :], axis=1) for i in range(NB)]
    out_ref[...] = jnp.stack(idxs, axis=0).astype(jnp.int32)[None]


def kernel(x, W1, b1, W2, b2, W3, b3, codebook):
    B, Cin, L = x.shape
    H = W1.shape[0]
    D = W3.shape[0]
    K = codebook.shape[0]
    kw1 = W1.shape[2]
    L2 = L // 4

    W1r = jnp.transpose(W1, (2, 1, 0)).reshape(Cin * kw1, H)
    W2r = jnp.transpose(W2, (2, 1, 0)).reshape(4 * H, H)
    W3r = jnp.transpose(W3, (2, 1, 0)).reshape(3 * H, D)
    Ct = codebook.T
    cn = jnp.sum(codebook * codebook, axis=1)[None, :]
    b1r, b2r, b3r = b1[None, :], b2[None, :], b3[None, :]

    out = pl.pallas_call(
        _encoder_kernel,
        grid=(B // _NB,),
        in_specs=[
            pl.BlockSpec((_NB, Cin, L), lambda b: (b, 0, 0)),
            pl.BlockSpec((Cin * kw1, H), lambda b: (0, 0)),
            pl.BlockSpec((1, H), lambda b: (0, 0)),
            pl.BlockSpec((4 * H, H), lambda b: (0, 0)),
            pl.BlockSpec((1, H), lambda b: (0, 0)),
            pl.BlockSpec((3 * H, D), lambda b: (0, 0)),
            pl.BlockSpec((1, D), lambda b: (0, 0)),
            pl.BlockSpec((D, K), lambda b: (0, 0)),
            pl.BlockSpec((1, K), lambda b: (0, 0)),
        ],
        out_specs=pl.BlockSpec((1, _NB, L2), lambda b: (b, 0, 0)),
        out_shape=jax.ShapeDtypeStruct((B // _NB, _NB, L2), jnp.int32),
        compiler_params=pltpu.CompilerParams(
            dimension_semantics=("arbitrary",)),
    )(x, W1r, b1r, W2r, b2r, W3r, b3r, Ct, cn)
    return out.reshape(B, L2)


# speedup vs baseline: 3.3870x; 1.0450x over previous
import jax
import jax.numpy as jnp
from jax.experimental import pallas as pl
from jax.experimental.pallas import tpu as pltpu

_NB = 8


def _tap_stack(xr):
    zc = jnp.zeros((xr.shape[0], 1), jnp.float32)
    t0 = jnp.concatenate([zc, xr[:, :-1]], axis=1)
    t2 = jnp.concatenate([xr[:, 1:], zc], axis=1)
    t3 = jnp.concatenate([xr[:, 2:], zc, zc], axis=1)
    return jnp.concatenate([t0, xr, t2, t3], axis=0)


def _encoder_kernel(x_ref, w1_ref, b1_ref, w2_ref, b2_ref, w3_ref, b3_ref,
                    ct_ref, cn_ref, out_ref):
    f32 = jnp.float32
    NB = x_ref.shape[0]
    CT = x_ref.shape[1] * 4
    L2 = x_ref.shape[2] // 4
    X = jnp.concatenate([_tap_stack(x_ref[i]) for i in range(NB)], axis=0)
    XT = X.T
    X4 = XT.reshape(XT.shape[0] // 4, 4, XT.shape[1])
    Xe, Xo = X4[:, 0, :], X4[:, 2, :]
    xcol = jnp.concatenate(
        [Xe[:, i * CT:(i + 1) * CT] for i in range(NB)]
        + [Xo[:, i * CT:(i + 1) * CT] for i in range(NB)], axis=0)
    h1 = jnp.dot(xcol, w1_ref[...], preferred_element_type=f32) + b1_ref[...]
    h1 = jnp.maximum(h1, 0.0)
    HE = h1[:NB * L2, :]
    HO = h1[NB * L2:, :]
    zrow = jnp.zeros((1, h1.shape[1]), f32)
    sro_parts, sle_parts = [], []
    for i in range(NB):
        sro_parts += [zrow, HO[i * L2:(i + 1) * L2 - 1, :]]
        sle_parts += [HE[i * L2 + 1:(i + 1) * L2, :], zrow]
    SRO = jnp.concatenate(sro_parts, axis=0)
    SLE = jnp.concatenate(sle_parts, axis=0)
    H = h1.shape[1]
    w2 = w2_ref[...]
    h2 = (jnp.dot(SRO, w2[:H], preferred_element_type=f32)
          + jnp.dot(HE, w2[H:2 * H], preferred_element_type=f32)
          + jnp.dot(HO, w2[2 * H:3 * H], preferred_element_type=f32)
          + jnp.dot(SLE, w2[3 * H:], preferred_element_type=f32)) + b2_ref[...]
    h2 = jnp.maximum(h2, 0.0)
    sr2_parts, sl2_parts = [], []
    for i in range(NB):
        sr2_parts += [zrow, h2[i * L2:(i + 1) * L2 - 1, :]]
        sl2_parts += [h2[i * L2 + 1:(i + 1) * L2, :], zrow]
    SR2 = jnp.concatenate(sr2_parts, axis=0)
    SL2 = jnp.concatenate(sl2_parts, axis=0)
    w3 = w3_ref[...]
    z = (jnp.dot(SR2, w3[:H], preferred_element_type=f32)
         + jnp.dot(h2, w3[H:2 * H], preferred_element_type=f32)
         + jnp.dot(SL2, w3[2 * H:], preferred_element_type=f32)) + b3_ref[...]
    d = cn_ref[...] - 2.0 * jnp.dot(z, ct_ref[...], preferred_element_type=f32)
    idxs = [jnp.argmin(d[i * L2:(i + 1) * L2, :], axis=1) for i in range(NB)]
    out_ref[...] = jnp.stack(idxs, axis=0).astype(jnp.int32)[None]


def kernel(x, W1, b1, W2, b2, W3, b3, codebook):
    B, Cin, L = x.shape
    H = W1.shape[0]
    D = W3.shape[0]
    K = codebook.shape[0]
    kw1 = W1.shape[2]
    L2 = L // 4

    W1r = jnp.transpose(W1, (2, 1, 0)).reshape(Cin * kw1, H)
    W2r = jnp.transpose(W2, (2, 1, 0)).reshape(4 * H, H)
    W3r = jnp.transpose(W3, (2, 1, 0)).reshape(3 * H, D)
    Ct = codebook.T
    cn = jnp.sum(codebook * codebook, axis=1)[None, :]
    b1r, b2r, b3r = b1[None, :], b2[None, :], b3[None, :]

    out = pl.pallas_call(
        _encoder_kernel,
        grid=(B // _NB,),
        in_specs=[
            pl.BlockSpec((_NB, Cin, L), lambda b: (b, 0, 0)),
            pl.BlockSpec((Cin * kw1, H), lambda b: (0, 0)),
            pl.BlockSpec((1, H), lambda b: (0, 0)),
            pl.BlockSpec((4 * H, H), lambda b: (0, 0)),
            pl.BlockSpec((1, H), lambda b: (0, 0)),
            pl.BlockSpec((3 * H, D), lambda b: (0, 0)),
            pl.BlockSpec((1, D), lambda b: (0, 0)),
            pl.BlockSpec((D, K), lambda b: (0, 0)),
            pl.BlockSpec((1, K), lambda b: (0, 0)),
        ],
        out_specs=pl.BlockSpec((1, _NB, L2), lambda b: (b, 0, 0)),
        out_shape=jax.ShapeDtypeStruct((B // _NB, _NB, L2), jnp.int32),
        compiler_params=pltpu.CompilerParams(
            dimension_semantics=("arbitrary",)),
    )(x, W1r, b1r, W2r, b2r, W3r, b3r, Ct, cn)
    return out.reshape(B, L2)


# speedup vs baseline: 3.3960x; 1.0027x over previous
import jax
import jax.numpy as jnp
from jax.experimental import pallas as pl
from jax.experimental.pallas import tpu as pltpu

_NB = 8


def _tap_stack(xr):
    zc = jnp.zeros((xr.shape[0], 1), jnp.float32)
    t0 = jnp.concatenate([zc, xr[:, :-1]], axis=1)
    t2 = jnp.concatenate([xr[:, 1:], zc], axis=1)
    t3 = jnp.concatenate([xr[:, 2:], zc, zc], axis=1)
    return jnp.concatenate([t0, xr, t2, t3], axis=0)


def _encoder_kernel(x_ref, w1_ref, b1_ref, w2_ref, b2_ref, w3_ref, b3_ref,
                    ct_ref, cn_ref, out_ref):
    f32 = jnp.float32
    NB = x_ref.shape[0]
    CT = x_ref.shape[1] * 4
    L2 = x_ref.shape[2] // 4
    X = jnp.concatenate([_tap_stack(x_ref[i]) for i in range(NB)], axis=0)
    XT = X.T
    X4 = XT.reshape(XT.shape[0] // 4, 4, XT.shape[1])
    Xe, Xo = X4[:, 0, :], X4[:, 2, :]
    xcol = jnp.concatenate(
        [Xe[:, i * CT:(i + 1) * CT] for i in range(NB)]
        + [Xo[:, i * CT:(i + 1) * CT] for i in range(NB)], axis=0)
    h1 = jnp.dot(xcol, w1_ref[...], preferred_element_type=f32) + b1_ref[...]
    h1 = jnp.maximum(h1, 0.0)
    HE = h1[:NB * L2, :]
    HO = h1[NB * L2:, :]
    zrow = jnp.zeros((1, h1.shape[1]), f32)
    sro_parts, sle_parts = [], []
    for i in range(NB):
        sro_parts += [zrow, HO[i * L2:(i + 1) * L2 - 1, :]]
        sle_parts += [HE[i * L2 + 1:(i + 1) * L2, :], zrow]
    SRO = jnp.concatenate(sro_parts, axis=0)
    SLE = jnp.concatenate(sle_parts, axis=0)
    H = h1.shape[1]
    w2 = w2_ref[...]
    h2 = (jnp.dot(SRO, w2[:H], preferred_element_type=f32)
          + jnp.dot(HE, w2[H:2 * H], preferred_element_type=f32)
          + jnp.dot(HO, w2[2 * H:3 * H], preferred_element_type=f32)
          + jnp.dot(SLE, w2[3 * H:], preferred_element_type=f32)) + b2_ref[...]
    h2 = jnp.maximum(h2, 0.0)
    sr2_parts, sl2_parts = [], []
    for i in range(NB):
        sr2_parts += [zrow, h2[i * L2:(i + 1) * L2 - 1, :]]
        sl2_parts += [h2[i * L2 + 1:(i + 1) * L2, :], zrow]
    SR2 = jnp.concatenate(sr2_parts, axis=0)
    SL2 = jnp.concatenate(sl2_parts, axis=0)
    w3 = w3_ref[...]
    z = (jnp.dot(SR2, w3[:H], preferred_element_type=f32)
         + jnp.dot(h2, w3[H:2 * H], preferred_element_type=f32)
         + jnp.dot(SL2, w3[2 * H:], preferred_element_type=f32)) + b3_ref[...]
    d = cn_ref[...] + jnp.dot(z, ct_ref[...], preferred_element_type=f32)
    idxs = [jnp.argmin(d[i * L2:(i + 1) * L2, :], axis=1) for i in range(NB)]
    out_ref[...] = jnp.stack(idxs, axis=0).astype(jnp.int32)[None]


def kernel(x, W1, b1, W2, b2, W3, b3, codebook):
    B, Cin, L = x.shape
    H = W1.shape[0]
    D = W3.shape[0]
    K = codebook.shape[0]
    kw1 = W1.shape[2]
    L2 = L // 4

    W1r = jnp.transpose(W1, (2, 1, 0)).reshape(Cin * kw1, H)
    W2r = jnp.transpose(W2, (2, 1, 0)).reshape(4 * H, H)
    W3r = jnp.transpose(W3, (2, 1, 0)).reshape(3 * H, D)
    Ct = -2.0 * codebook.T
    cn = jnp.sum(codebook * codebook, axis=1)[None, :]
    b1r, b2r, b3r = b1[None, :], b2[None, :], b3[None, :]

    out = pl.pallas_call(
        _encoder_kernel,
        grid=(B // _NB,),
        in_specs=[
            pl.BlockSpec((_NB, Cin, L), lambda b: (b, 0, 0)),
            pl.BlockSpec((Cin * kw1, H), lambda b: (0, 0)),
            pl.BlockSpec((1, H), lambda b: (0, 0)),
            pl.BlockSpec((4 * H, H), lambda b: (0, 0)),
            pl.BlockSpec((1, H), lambda b: (0, 0)),
            pl.BlockSpec((3 * H, D), lambda b: (0, 0)),
            pl.BlockSpec((1, D), lambda b: (0, 0)),
            pl.BlockSpec((D, K), lambda b: (0, 0)),
            pl.BlockSpec((1, K), lambda b: (0, 0)),
        ],
        out_specs=pl.BlockSpec((1, _NB, L2), lambda b: (b, 0, 0)),
        out_shape=jax.ShapeDtypeStruct((B // _NB, _NB, L2), jnp.int32),
        compiler_params=pltpu.CompilerParams(
            dimension_semantics=("arbitrary",)),
    )(x, W1r, b1r, W2r, b2r, W3r, b3r, Ct, cn)
    return out.reshape(B, L2)
